# Initial kernel scaffold; baseline (speedup 1.0000x reference)
import jax
import jax.numpy as jnp
from jax.experimental import pallas as pl


def kernel(features, edge_index, segment_ids, W1, W2, W3, b3, W4, b4):
    raise NotImplementedError("write your pallas kernel here")


# speedup vs baseline: 11.3123x; 11.3123x over previous
import functools

import jax
import jax.numpy as jnp
from jax import lax
from jax.experimental import pallas as pl
from jax.experimental.pallas import tpu as pltpu
from jax.experimental.pallas import tpu_sc as plsc

_N = 10000
_E = 320000
_D = 128
_H1 = 128
_H2 = 64
_H3 = 64
_L = 32
_G = 64

_NC = 2
_NS = 16
_NW = _NC * _NS
_EW = _E // _NW
_B = 80
_NBATCH = _EW // _B
_CHUNK = 632

_BLK = 1000

_sc_mesh = plsc.VectorSubcoreMesh(
    core_axis_name="c", subcore_axis_name="s", num_cores=_NC, num_subcores=_NS
)


def _make_deg_kernel():
  @functools.partial(
      pl.kernel,
      mesh=_sc_mesh,
      compiler_params=pltpu.CompilerParams(use_tc_tiling_on_sc=False),
      out_type=jax.ShapeDtypeStruct((_NC, _N, 16), jnp.float32),
      scratch_types=[
          pltpu.VMEM((_B,), jnp.int32),
          pltpu.VMEM((_B, 16), jnp.float32),
          pltpu.VMEM_SHARED((_N, 16), jnp.float32),
      ],
  )
  def deg_kernel(dst_hbm, ones_hbm, zeros_hbm, out_hbm, dst_v, ones_v, acc_sh):
    c = lax.axis_index("c")
    s = lax.axis_index("s")
    off = pl.multiple_of(jnp.minimum(s * _CHUNK, _N - _CHUNK), 8)
    pltpu.sync_copy(zeros_hbm, acc_sh.at[pl.ds(off, _CHUNK)])
    pltpu.sync_copy(ones_hbm, ones_v)
    plsc.subcore_barrier()
    base = (c * _NS + s) * _EW

    @pl.loop(0, _NBATCH)
    def _(b):
      pltpu.sync_copy(dst_hbm.at[pl.ds(base + b * _B, _B)], dst_v)
      pltpu.sync_copy(ones_v, acc_sh.at[dst_v], add=True)

    plsc.subcore_barrier()
    pltpu.sync_copy(acc_sh.at[pl.ds(off, _CHUNK)],
                    out_hbm.at[c, pl.ds(off, _CHUNK)])

  return deg_kernel


def _make_agg_kernel(h):
  params = None if h % 128 == 0 else pltpu.CompilerParams(
      use_tc_tiling_on_sc=False)

  @functools.partial(
      pl.kernel,
      mesh=_sc_mesh,
      compiler_params=params,
      out_type=jax.ShapeDtypeStruct((_NC, _N, h), jnp.float32),
      scratch_types=[
          pltpu.VMEM((_B,), jnp.int32),
          pltpu.VMEM((_B,), jnp.int32),
          pltpu.VMEM((_B, h), jnp.float32),
          pltpu.VMEM_SHARED((_N, h), jnp.float32),
          pltpu.SemaphoreType.DMA,
      ],
  )
  def agg_kernel(hwp_hbm, src_hbm, dst_hbm, zeros_hbm, out_hbm,
                 src_v, dst_v, rows_v, acc_sh, sem):
    c = lax.axis_index("c")
    s = lax.axis_index("s")
    off = pl.multiple_of(jnp.minimum(s * _CHUNK, _N - _CHUNK), 8)
    pltpu.sync_copy(zeros_hbm, acc_sh.at[pl.ds(off, _CHUNK)])
    plsc.subcore_barrier()
    base = (c * _NS + s) * _EW

    @pl.loop(0, _NBATCH)
    def _(b):
      pltpu.sync_copy(src_hbm.at[pl.ds(base + b * _B, _B)], src_v)
      pltpu.sync_copy(dst_hbm.at[pl.ds(base + b * _B, _B)], dst_v)
      pltpu.async_copy(hwp_hbm.at[src_v], rows_v, sem).wait()
      pltpu.sync_copy(rows_v, acc_sh.at[dst_v], add=True)

    plsc.subcore_barrier()
    pltpu.sync_copy(acc_sh.at[pl.ds(off, _CHUNK)],
                    out_hbm.at[c, pl.ds(off, _CHUNK)])

  return agg_kernel


_deg_call = _make_deg_kernel()
_agg_call_128 = _make_agg_kernel(_H1)
_agg_call_64 = _make_agg_kernel(_H2)


def _isd_from_deg(deg_ref):
  d = deg_ref[0, :, 0:1] + deg_ref[1, :, 0:1] + 1.0
  return lax.rsqrt(d)


def _mm1_body(feat_ref, w_ref, deg_ref, out_ref):
  isd = _isd_from_deg(deg_ref)
  hw = jnp.dot(feat_ref[...], w_ref[...], preferred_element_type=jnp.float32)
  out_ref[...] = hw * isd


def _comb_mm_body(agg_ref, hwp_ref, deg_ref, w_ref, out_ref):
  isd = _isd_from_deg(deg_ref)
  h = jnp.maximum((agg_ref[0, :, :] + agg_ref[1, :, :] + hwp_ref[...]) * isd,
                  0.0)
  out_ref[...] = jnp.dot(h, w_ref[...],
                         preferred_element_type=jnp.float32) * isd


def _pool_body(agg_ref, hwp_ref, deg_ref, seg_ref, sums_ref, counts_ref):
  i = pl.program_id(0)
  isd = _isd_from_deg(deg_ref)
  h2 = jnp.maximum((agg_ref[0, :, :] + agg_ref[1, :, :] + hwp_ref[...]) * isd,
                   0.0)
  seg = seg_ref[0, 0, :]
  onehot = (lax.broadcasted_iota(jnp.int32, (_G, _BLK), 0)
            == seg[None, :]).astype(jnp.float32)
  part = jnp.dot(onehot, h2, preferred_element_type=jnp.float32)
  cnt = jnp.sum(onehot, axis=1, keepdims=True) * jnp.ones((1, _H2),
                                                          jnp.float32)

  @pl.when(i == 0)
  def _():
    sums_ref[...] = jnp.zeros_like(sums_ref)
    counts_ref[...] = jnp.zeros_like(counts_ref)

  sums_ref[...] += part
  counts_ref[...] += cnt


def _head_body(sums_ref, counts_ref, w3_ref, b3_ref, w4_ref, b4_ref, out_ref):
  pooled = sums_ref[...] / jnp.maximum(counts_ref[...], 1.0)
  h3 = jnp.maximum(
      jnp.dot(pooled, w3_ref[...], preferred_element_type=jnp.float32)
      + b3_ref[...], 0.0)
  out_ref[...] = (jnp.dot(h3, w4_ref[...], preferred_element_type=jnp.float32)
                  + b4_ref[...])


def kernel(features, edge_index, segment_ids, W1, W2, W3, b3, W4, b4):
  src = edge_index[0]
  dst = edge_index[1]

  ones16 = jnp.ones((_B, 16), jnp.float32)
  zeros16 = jnp.zeros((_CHUNK, 16), jnp.float32)
  zeros128 = jnp.zeros((_CHUNK, _H1), jnp.float32)
  zeros64 = jnp.zeros((_CHUNK, _H2), jnp.float32)

  deg16 = _deg_call(dst, ones16, zeros16)

  grid = (_N // _BLK,)
  deg_spec = pl.BlockSpec((_NC, _BLK, 16), lambda i: (0, i, 0))

  hwp1 = pl.pallas_call(
      _mm1_body,
      grid=grid,
      in_specs=[
          pl.BlockSpec((_BLK, _D), lambda i: (i, 0)),
          pl.BlockSpec((_D, _H1), lambda i: (0, 0)),
          deg_spec,
      ],
      out_specs=pl.BlockSpec((_BLK, _H1), lambda i: (i, 0)),
      out_shape=jax.ShapeDtypeStruct((_N, _H1), jnp.float32),
  )(features, W1, deg16)

  agg1 = _agg_call_128(hwp1, src, dst, zeros128)

  hwp2 = pl.pallas_call(
      _comb_mm_body,
      grid=grid,
      in_specs=[
          pl.BlockSpec((_NC, _BLK, _H1), lambda i: (0, i, 0)),
          pl.BlockSpec((_BLK, _H1), lambda i: (i, 0)),
          deg_spec,
          pl.BlockSpec((_H1, _H2), lambda i: (0, 0)),
      ],
      out_specs=pl.BlockSpec((_BLK, _H2), lambda i: (i, 0)),
      out_shape=jax.ShapeDtypeStruct((_N, _H2), jnp.float32),
  )(agg1, hwp1, deg16, W2)

  agg2 = _agg_call_64(hwp2, src, dst, zeros64)

  seg3d = segment_ids.reshape(_N // _BLK, 1, _BLK)
  sums, counts = pl.pallas_call(
      _pool_body,
      grid=grid,
      in_specs=[
          pl.BlockSpec((_NC, _BLK, _H2), lambda i: (0, i, 0)),
          pl.BlockSpec((_BLK, _H2), lambda i: (i, 0)),
          deg_spec,
          pl.BlockSpec((1, 1, _BLK), lambda i: (i, 0, 0)),
      ],
      out_specs=[
          pl.BlockSpec((_G, _H2), lambda i: (0, 0)),
          pl.BlockSpec((_G, _H2), lambda i: (0, 0)),
      ],
      out_shape=[
          jax.ShapeDtypeStruct((_G, _H2), jnp.float32),
          jax.ShapeDtypeStruct((_G, _H2), jnp.float32),
      ],
  )(agg2, hwp2, deg16, seg3d)

  out = pl.pallas_call(
      _head_body,
      in_specs=[
          pl.BlockSpec((_G, _H2), lambda: (0, 0)),
          pl.BlockSpec((_G, _H2), lambda: (0, 0)),
          pl.BlockSpec((_H2, _H3), lambda: (0, 0)),
          pl.BlockSpec((1, _H3), lambda: (0, 0)),
          pl.BlockSpec((_H3, _L), lambda: (0, 0)),
          pl.BlockSpec((1, _L), lambda: (0, 0)),
      ],
      out_specs=pl.BlockSpec((_G, _L), lambda: (0, 0)),
      out_shape=jax.ShapeDtypeStruct((_G, _L), jnp.float32),
  )(sums, counts, W3, b3.reshape(1, _H3), W4, b4.reshape(1, _L))

  return out
